# baseline (device time: 49724 ns/iter reference)
import functools

import jax
import jax.numpy as jnp
from jax import lax
from jax.experimental import pallas as pl
from jax.experimental.pallas import tpu as pltpu

N_DEV = 8
B = 2
SQ = 128
SKV = 128
D = 512
H = 8
DH = 64
SCALE = 0.125


def kernel(x, Wq, Wo, K_ext, V_ext):
    xb = x.astype(jnp.bfloat16)
    wqb = Wq.astype(jnp.bfloat16)
    wob = Wo.astype(jnp.bfloat16)
    kb = K_ext.reshape(B, SKV, D).astype(jnp.bfloat16)
    vb = V_ext.reshape(B, SKV, D).astype(jnp.bfloat16)

    def body(x_ref, wq_ref, wo_ref, k_ref, v_ref, out_ref,
             kv_full, q_scr, attn_scr, send_sems, recv_sems):
        my = lax.axis_index("i")

        barrier_sem = pltpu.get_barrier_semaphore()
        for r in range(1, N_DEV):
            pl.semaphore_signal(
                barrier_sem, inc=1,
                device_id=(lax.rem(my + r, N_DEV),),
                device_id_type=pl.DeviceIdType.MESH,
            )
        pl.semaphore_wait(barrier_sem, N_DEV - 1)

        kv_full[0, :, pl.ds(my * SKV, SKV), :] = k_ref[...]
        kv_full[1, :, pl.ds(my * SKV, SKV), :] = v_ref[...]

        sends = []
        for r in range(1, N_DEV):
            c = pltpu.make_async_remote_copy(
                src_ref=kv_full.at[:, :, pl.ds(my * SKV, SKV), :],
                dst_ref=kv_full.at[:, :, pl.ds(my * SKV, SKV), :],
                send_sem=send_sems.at[r - 1],
                recv_sem=recv_sems.at[r - 1],
                device_id=(lax.rem(my + r, N_DEV),),
                device_id_type=pl.DeviceIdType.MESH,
            )
            c.start()
            sends.append(c)

        for b in range(B):
            q_scr[b] = jnp.dot(
                x_ref[b], wq_ref[...], preferred_element_type=jnp.float32
            ).astype(jnp.bfloat16)

        for s in range(1, N_DEV):
            o = lax.rem(my - s + N_DEV, N_DEV)
            recv = pltpu.make_async_remote_copy(
                src_ref=kv_full.at[:, :, pl.ds(o * SKV, SKV), :],
                dst_ref=kv_full.at[:, :, pl.ds(o * SKV, SKV), :],
                send_sem=send_sems.at[s - 1],
                recv_sem=recv_sems.at[s - 1],
                device_id=(o,),
                device_id_type=pl.DeviceIdType.MESH,
            )
            recv.wait_recv()
        for c in sends:
            c.wait_send()

        for b in range(B):
            for hh in range(H):
                q = q_scr[b, :, hh * DH:(hh + 1) * DH]
                kh = kv_full[0, b, :, hh * DH:(hh + 1) * DH]
                s = lax.dot_general(
                    q, kh, (((1,), (1,)), ((), ())),
                    preferred_element_type=jnp.float32,
                ) * SCALE
                m = jnp.max(s, axis=1, keepdims=True)
                p = jnp.exp(s - m)
                l = jnp.sum(p, axis=1, keepdims=True)
                vh = kv_full[1, b, :, hh * DH:(hh + 1) * DH]
                o = lax.dot_general(
                    p.astype(jnp.bfloat16), vh, (((1,), (0,)), ((), ())),
                    preferred_element_type=jnp.float32,
                )
                attn_scr[b, :, hh * DH:(hh + 1) * DH] = (o / l).astype(
                    jnp.bfloat16
                )

        for b in range(B):
            out_ref[b] = jnp.dot(
                attn_scr[b], wo_ref[...], preferred_element_type=jnp.float32
            )

        @functools.partial(
            pl.run_scoped, second_barrier=pltpu.SemaphoreType.REGULAR
        )
        def _(second_barrier):
            for r in range(1, N_DEV):
                pl.semaphore_signal(
                    second_barrier, inc=1,
                    device_id=(lax.rem(my + r, N_DEV),),
                    device_id_type=pl.DeviceIdType.MESH,
                )
            pl.semaphore_wait(second_barrier, N_DEV - 1)

    return pl.pallas_call(
        body,
        out_shape=jax.ShapeDtypeStruct((B, SQ, D), jnp.float32),
        in_specs=[pl.BlockSpec(memory_space=pltpu.VMEM)] * 5,
        out_specs=pl.BlockSpec(memory_space=pltpu.VMEM),
        scratch_shapes=[
            pltpu.VMEM((2, B, N_DEV * SKV, D), jnp.bfloat16),
            pltpu.VMEM((B, SQ, D), jnp.bfloat16),
            pltpu.VMEM((B, SQ, D), jnp.bfloat16),
            pltpu.SemaphoreType.DMA((N_DEV - 1,)),
            pltpu.SemaphoreType.DMA((N_DEV - 1,)),
        ],
        compiler_params=pltpu.CompilerParams(collective_id=0),
    )(xb, wqb, wob, kb, vb)
